# Initial kernel scaffold; baseline (speedup 1.0000x reference)
import jax
import jax.numpy as jnp
from jax.experimental import pallas as pl


def kernel(seq, idx, bond_feats, dist_matrix, emb_res_W, emb_atom_W):
    raise NotImplementedError("write your pallas kernel here")


# speedup vs baseline: 10.9306x; 10.9306x over previous
import functools

import jax
import jax.numpy as jnp
from jax import lax
from jax.experimental import pallas as pl
from jax.experimental.pallas import tpu as pltpu
from jax.experimental.pallas import tpu_sc as plsc

L = 1024
DP = 64
NCORES = 2
NSUB = 16
NW = NCORES * NSUB
ROWS_PER_W = L // NW
CHUNK = 128
NCH = L // CHUNK
A_PER_SUB = 5
TAB_ROWS = 16 * A_PER_SUB * 10


def _pe2d_body(seq_hbm, idx_hbm, dist_hbm, wres_hbm, wat_hbm,
               out_hbm, tab_hbm,
               seq_v, idx_v, wres_v, wat_v, row_v, dist_v, cidx_v, rows_v,
               sem_g, sem_o):
    cid = lax.axis_index("c")
    sid = lax.axis_index("s")
    wid = cid * NSUB + sid

    pltpu.sync_copy(seq_hbm, seq_v.at[pl.ds(0, L)])
    pltpu.sync_copy(idx_hbm, idx_v.at[pl.ds(0, L)])
    pltpu.sync_copy(wres_hbm, wres_v)
    pltpu.sync_copy(wat_hbm, wat_v)

    def _mask_body(j, _):
        s = seq_v[pl.ds(j * 16, 16)]
        seq_v[pl.ds(j * 16, 16)] = jnp.where(s >= 32, 1, 0).astype(jnp.int32)
        return 0
    lax.fori_loop(0, L // 16, _mask_body, 0)

    def _tab_a(aloc, _):
        a = sid * A_PER_SUB + aloc
        ac = jnp.minimum(a, 65)

        def _tab_b(b, _):
            for q in range(DP // 16):
                av = wres_v[pl.ds(ac * DP + q * 16, 16)]
                bv = wat_v[pl.ds(b * DP + q * 16, 16)]
                row_v[pl.ds(q * 16, 16)] = av + bv
            pltpu.sync_copy(row_v, tab_hbm.at[cid * TAB_ROWS + a * 10 + b])
            return 0
        lax.fori_loop(0, 10, _tab_b, 0)
        return 0
    lax.fori_loop(0, A_PER_SUB, _tab_a, 0)
    plsc.subcore_barrier()

    tbase = cid * TAB_ROWS

    def _row(iloc, _):
        i = wid * ROWS_PER_W + iloc
        pltpu.sync_copy(dist_hbm.at[i], dist_v)
        idxi = jnp.full((16,), idx_v[pl.ds(i, 16)][0], jnp.int32)
        smi = jnp.full((16,), seq_v[pl.ds(i, 16)][0], jnp.int32)

        def _chunk(k, _):
            def _cidx(j2, _):
                off = k * CHUNK + j2 * 16
                idxj = idx_v[pl.ds(off, 16)]
                smj = seq_v[pl.ds(off, 16)]
                dv = dist_v[pl.ds(off, 16)]
                rd = jnp.clip(idxj - idxi, -32, 33)
                ib_res = jnp.where(smi + smj > 0, 33, rd) + 32
                ad = jnp.minimum(jnp.maximum(dv, 0.0), 9.0)
                ad = jnp.where(smi + smj == 2, ad, 9.0)
                t = ad.astype(jnp.int32)
                ib_atom = t + jnp.where(t.astype(jnp.float32) < ad, 1, 0)
                cidx_v[pl.ds(j2 * 16, 16)] = tbase + ib_res * 10 + ib_atom
                return 0
            lax.fori_loop(0, CHUNK // 16, _cidx, 0)
            pltpu.async_copy(tab_hbm.at[cidx_v], rows_v, sem_g).wait()
            pltpu.async_copy(
                rows_v, out_hbm.at[pl.ds(i * L + k * CHUNK, CHUNK)], sem_o
            ).wait()
            return 0
        lax.fori_loop(0, NCH, _chunk, 0)
        return 0
    lax.fori_loop(0, ROWS_PER_W, _row, 0)


@jax.jit
def _pe2d(seq1, idx1, dist, wres, wat):
    mesh = plsc.VectorSubcoreMesh(core_axis_name="c", subcore_axis_name="s")
    f = pl.kernel(
        _pe2d_body,
        out_type=(
            jax.ShapeDtypeStruct((L * L, DP), jnp.float32),
            jax.ShapeDtypeStruct((NCORES * TAB_ROWS, DP), jnp.float32),
        ),
        mesh=mesh,
        scratch_types=[
            pltpu.VMEM((L + 16,), jnp.int32),
            pltpu.VMEM((L + 16,), jnp.int32),
            pltpu.VMEM((66 * DP,), jnp.float32),
            pltpu.VMEM((10 * DP,), jnp.float32),
            pltpu.VMEM((DP,), jnp.float32),
            pltpu.VMEM((L,), jnp.float32),
            pltpu.VMEM((CHUNK,), jnp.int32),
            pltpu.VMEM((CHUNK, DP), jnp.float32),
            pltpu.SemaphoreType.DMA,
            pltpu.SemaphoreType.DMA,
        ],
        compiler_params=pltpu.CompilerParams(use_tc_tiling_on_sc=False),
    )
    out, _tab = f(seq1, idx1, dist, wres, wat)
    return out


def kernel(seq, idx, bond_feats, dist_matrix, emb_res_W, emb_atom_W):
    del bond_feats
    seq1 = seq.reshape(L).astype(jnp.int32)
    idx1 = idx.reshape(L).astype(jnp.int32)
    dist = dist_matrix.reshape(L, L)
    wres = emb_res_W.reshape(66 * DP)
    wat = emb_atom_W.reshape(10 * DP)
    out = _pe2d(seq1, idx1, dist, wres, wat)
    return out.reshape(1, L, L, DP)
